# baseline (speedup 1.0000x reference)
import jax
import jax.numpy as jnp
from jax import lax
from jax.experimental import pallas as pl
from jax.experimental.pallas import tpu as pltpu
from jax.experimental.pallas import tpu_sc as plsc

B = 16384
D = 32
FD = 16
NROWS = 1000000
BLK = 32768
PADN = 1015808
GA = PADN // BLK

NC = 2
NS = 16
NW = NC * NS
BPW = B // NW
CHUNK = 128
NCHUNK = BPW // CHUNK


def _pv_u_body(wout, ut, uft, ift, wuf, wif, buf, bif, bo, pu, f):
    f32 = jnp.float32
    wu = wout[:, :D]
    wi = wout[:, D:]
    pu[...] = jnp.dot(wu, ut[...], preferred_element_type=f32).reshape(BLK)

    @pl.when(pl.program_id(0) == 0)
    def _():
        qu = lax.dot_general(wuf[...], wu, (((1,), (1,)), ((), ())),
                             preferred_element_type=f32)
        qi = lax.dot_general(wif[...], wi, (((1,), (1,)), ((), ())),
                             preferred_element_type=f32)
        fu = lax.dot_general(qu, uft[...], (((0,), (0,)), ((), ())),
                             preferred_element_type=f32)
        fi = lax.dot_general(qi, ift[...], (((0,), (0,)), ((), ())),
                             preferred_element_type=f32)
        const = (jnp.sum(buf[...] * wu) + jnp.sum(bif[...] * wi) + bo[0, 0])
        f[...] = fu + fi + const


_pv_u = pl.pallas_call(
    _pv_u_body,
    grid=(GA,),
    in_specs=[
        pl.BlockSpec((1, 2 * D), lambda j: (0, 0)),
        pl.BlockSpec((D, BLK), lambda j: (0, j)),
        pl.BlockSpec((FD, B), lambda j: (0, 0)),
        pl.BlockSpec((FD, B), lambda j: (0, 0)),
        pl.BlockSpec((FD, D), lambda j: (0, 0)),
        pl.BlockSpec((FD, D), lambda j: (0, 0)),
        pl.BlockSpec((1, D), lambda j: (0, 0)),
        pl.BlockSpec((1, D), lambda j: (0, 0)),
        pl.BlockSpec((1, 1), lambda j: (0, 0)),
    ],
    out_specs=[
        pl.BlockSpec((BLK,), lambda j: (j,)),
        pl.BlockSpec((1, B), lambda j: (0, 0)),
    ],
    out_shape=[
        jax.ShapeDtypeStruct((PADN,), jnp.float32),
        jax.ShapeDtypeStruct((1, B), jnp.float32),
    ],
)


def _pv_i_body(wout, it, pi):
    pi[...] = jnp.dot(wout[:, D:], it[...],
                      preferred_element_type=jnp.float32).reshape(BLK)


_pv_i = pl.pallas_call(
    _pv_i_body,
    grid=(GA,),
    in_specs=[
        pl.BlockSpec((1, 2 * D), lambda j: (0, 0)),
        pl.BlockSpec((D, BLK), lambda j: (0, j)),
    ],
    out_specs=pl.BlockSpec((BLK,), lambda j: (j,)),
    out_shape=jax.ShapeDtypeStruct((PADN,), jnp.float32),
)


def _gather_chunks(tab, idx, idxv, vals, sem, wid):
    pltpu.sync_copy(idx.at[pl.ds(wid * NCHUNK, NCHUNK)], idxv)
    for j in range(NCHUNK):
        pltpu.async_copy(tab.at[idxv.at[j]],
                         vals.at[pl.ds(j * CHUNK, CHUNK)], sem)
    for j in range(NCHUNK):
        pltpu.make_async_copy(tab.at[idxv.at[j]],
                              vals.at[pl.ds(j * CHUNK, CHUNK)], sem).wait()


def _sc_u_body(pu, idxu, su, idxv, vals, sem):
    wid = lax.axis_index("s") * NC + lax.axis_index("c")
    _gather_chunks(pu, idxu, idxv, vals, sem, wid)
    pltpu.sync_copy(vals, su.at[pl.ds(wid * BPW, BPW)])


_sc_u = pl.kernel(
    _sc_u_body,
    out_type=jax.ShapeDtypeStruct((B,), jnp.float32),
    mesh=plsc.VectorSubcoreMesh(core_axis_name="c", subcore_axis_name="s",
                                num_cores=NC, num_subcores=NS),
    scratch_types=[
        pltpu.VMEM((NCHUNK, CHUNK), jnp.int32),
        pltpu.VMEM((BPW,), jnp.float32),
        pltpu.SemaphoreType.DMA,
    ],
)


def _sc_i_body(pi, su, f, idxi, out, idxv, vals, suv, fv, sem):
    wid = lax.axis_index("s") * NC + lax.axis_index("c")
    base = wid * BPW
    pltpu.sync_copy(idxi.at[pl.ds(wid * NCHUNK, NCHUNK)], idxv)
    for j in range(NCHUNK):
        pltpu.async_copy(pi.at[idxv.at[j]],
                         vals.at[pl.ds(j * CHUNK, CHUNK)], sem)
    pltpu.sync_copy(su.at[pl.ds(base, BPW)], suv)
    pltpu.sync_copy(f.at[pl.ds(base, BPW)], fv)
    for j in range(NCHUNK):
        pltpu.make_async_copy(pi.at[idxv.at[j]],
                              vals.at[pl.ds(j * CHUNK, CHUNK)], sem).wait()
    vals[...] = vals[...] + suv[...] + fv[...]
    pltpu.sync_copy(vals, out.at[pl.ds(base, BPW)])


_sc_i = pl.kernel(
    _sc_i_body,
    out_type=jax.ShapeDtypeStruct((B,), jnp.float32),
    mesh=plsc.VectorSubcoreMesh(core_axis_name="c", subcore_axis_name="s",
                                num_cores=NC, num_subcores=NS),
    scratch_types=[
        pltpu.VMEM((NCHUNK, CHUNK), jnp.int32),
        pltpu.VMEM((BPW,), jnp.float32),
        pltpu.VMEM((BPW,), jnp.float32),
        pltpu.VMEM((BPW,), jnp.float32),
        pltpu.SemaphoreType.DMA,
    ],
)


def kernel(user_ids, item_ids, user_features, item_features, user_emb,
           item_emb, W_uf, b_uf, W_if, b_if, W_out, b_out):
    wout = W_out.reshape(1, 2 * D)
    pu, f = _pv_u(wout, user_emb.T, user_features.T, item_features.T,
                  W_uf, W_if, b_uf.reshape(1, D), b_if.reshape(1, D),
                  b_out.reshape(1, 1))
    su = _sc_u(pu, user_ids.reshape(NW * NCHUNK, CHUNK))
    pi = _pv_i(wout, item_emb.T)
    out = _sc_i(pi, su, f.reshape(B),
                item_ids.reshape(NW * NCHUNK, CHUNK))
    return out.reshape(B, 1)


# speedup vs baseline: 1.1335x; 1.1335x over previous
import jax
import jax.numpy as jnp
from jax import lax
from jax.experimental import pallas as pl
from jax.experimental.pallas import tpu as pltpu
from jax.experimental.pallas import tpu_sc as plsc

B = 16384
D = 32
FD = 16
NROWS = 1000000
BLK = 32768
PADN = 1015808
GA = PADN // BLK

NC = 2
NS = 16
NW = NC * NS
BPW = B // NW
CHUNK = 128
NCHUNK = BPW // CHUNK


def _pv_body(wout, ut, it, uft, ift, wuf, wif, buf, bif, bo, pu, pi, f):
    f32 = jnp.float32
    wu = wout[:, :D]
    wi = wout[:, D:]
    pu[...] = jnp.dot(wu, ut[...], preferred_element_type=f32).reshape(BLK)
    pi[...] = jnp.dot(wi, it[...], preferred_element_type=f32).reshape(BLK)

    @pl.when(pl.program_id(0) == 0)
    def _():
        qu = lax.dot_general(wuf[...], wu, (((1,), (1,)), ((), ())),
                             preferred_element_type=f32)
        qi = lax.dot_general(wif[...], wi, (((1,), (1,)), ((), ())),
                             preferred_element_type=f32)
        fu = lax.dot_general(qu, uft[...], (((0,), (0,)), ((), ())),
                             preferred_element_type=f32)
        fi = lax.dot_general(qi, ift[...], (((0,), (0,)), ((), ())),
                             preferred_element_type=f32)
        const = (jnp.sum(buf[...] * wu) + jnp.sum(bif[...] * wi) + bo[0, 0])
        f[...] = fu + fi + const


_pv = pl.pallas_call(
    _pv_body,
    grid=(GA,),
    in_specs=[
        pl.BlockSpec((1, 2 * D), lambda j: (0, 0)),
        pl.BlockSpec((D, BLK), lambda j: (0, j)),
        pl.BlockSpec((D, BLK), lambda j: (0, j)),
        pl.BlockSpec((FD, B), lambda j: (0, 0)),
        pl.BlockSpec((FD, B), lambda j: (0, 0)),
        pl.BlockSpec((FD, D), lambda j: (0, 0)),
        pl.BlockSpec((FD, D), lambda j: (0, 0)),
        pl.BlockSpec((1, D), lambda j: (0, 0)),
        pl.BlockSpec((1, D), lambda j: (0, 0)),
        pl.BlockSpec((1, 1), lambda j: (0, 0)),
    ],
    out_specs=[
        pl.BlockSpec((BLK,), lambda j: (j,)),
        pl.BlockSpec((BLK,), lambda j: (j,)),
        pl.BlockSpec((1, B), lambda j: (0, 0)),
    ],
    out_shape=[
        jax.ShapeDtypeStruct((PADN,), jnp.float32),
        jax.ShapeDtypeStruct((PADN,), jnp.float32),
        jax.ShapeDtypeStruct((1, B), jnp.float32),
    ],
)


def _sc_body(pu, pi, f, idxu, idxi, out, idxvu, idxvi, valsu, valsi, fv,
             sem):
    wid = lax.axis_index("s") * NC + lax.axis_index("c")
    base = wid * BPW
    pltpu.sync_copy(idxu.at[pl.ds(wid * NCHUNK, NCHUNK)], idxvu)
    pltpu.sync_copy(idxi.at[pl.ds(wid * NCHUNK, NCHUNK)], idxvi)
    for j in range(NCHUNK):
        pltpu.async_copy(pu.at[idxvu.at[j]],
                         valsu.at[pl.ds(j * CHUNK, CHUNK)], sem)
    for j in range(NCHUNK):
        pltpu.async_copy(pi.at[idxvi.at[j]],
                         valsi.at[pl.ds(j * CHUNK, CHUNK)], sem)
    pltpu.sync_copy(f.at[pl.ds(base, BPW)], fv)
    for j in range(NCHUNK):
        pltpu.make_async_copy(pu.at[idxvu.at[j]],
                              valsu.at[pl.ds(j * CHUNK, CHUNK)], sem).wait()
    for j in range(NCHUNK):
        pltpu.make_async_copy(pi.at[idxvi.at[j]],
                              valsi.at[pl.ds(j * CHUNK, CHUNK)], sem).wait()
    valsu[...] = valsu[...] + valsi[...] + fv[...]
    pltpu.sync_copy(valsu, out.at[pl.ds(base, BPW)])


_sc_gather = pl.kernel(
    _sc_body,
    out_type=jax.ShapeDtypeStruct((B,), jnp.float32),
    mesh=plsc.VectorSubcoreMesh(core_axis_name="c", subcore_axis_name="s",
                                num_cores=NC, num_subcores=NS),
    scratch_types=[
        pltpu.VMEM((NCHUNK, CHUNK), jnp.int32),
        pltpu.VMEM((NCHUNK, CHUNK), jnp.int32),
        pltpu.VMEM((BPW,), jnp.float32),
        pltpu.VMEM((BPW,), jnp.float32),
        pltpu.VMEM((BPW,), jnp.float32),
        pltpu.SemaphoreType.DMA,
    ],
)


def kernel(user_ids, item_ids, user_features, item_features, user_emb,
           item_emb, W_uf, b_uf, W_if, b_if, W_out, b_out):
    wout = W_out.reshape(1, 2 * D)
    pu, pi, f = _pv(wout, user_emb.T, item_emb.T,
                    user_features.T, item_features.T, W_uf, W_if,
                    b_uf.reshape(1, D), b_if.reshape(1, D),
                    b_out.reshape(1, 1))
    idxu = user_ids.reshape(NW * NCHUNK, CHUNK)
    idxi = item_ids.reshape(NW * NCHUNK, CHUNK)
    out = _sc_gather(pu, pi, f.reshape(B), idxu, idxi)
    return out.reshape(B, 1)
